# baseline (speedup 1.0000x reference)
import functools

import jax
import jax.numpy as jnp
from jax import lax
from jax.experimental import pallas as pl
from jax.experimental.pallas import tpu as pltpu
from jax.experimental.pallas import tpu_sc as plsc

B = 16384
ED = 10
NC, NS = 2, 16
NW = NC * NS
BPW = B // NW
CHUNK = 128
NCH = BPW // CHUNK


CW = 8192


def _vp(v0):
    return -(-v0 // 128) * 128


def _detile(tts, tails):
    mesh = plsc.VectorSubcoreMesh(core_axis_name="c", subcore_axis_name="s")
    v0s = [t.shape[1] - 1 for t in tts]
    out_t = [jax.ShapeDtypeStruct((ED * _vp(v0),), jnp.float32) for v0 in v0s]

    @functools.partial(pl.kernel, out_type=out_t, mesh=mesh,
                       scratch_types=[pltpu.VMEM((ED, CW), jnp.float32)],
                       compiler_params=pltpu.CompilerParams(
                           use_tc_tiling_on_sc=True))
    def k(t0, t1, t2, t3, t4, a0, a1, a2, a3, a4,
          o0, o1, o2, o3, o4, buf):
        wid = lax.axis_index("s") * NC + lax.axis_index("c")
        for t, (tab, ta, out, v0) in enumerate(zip(
                (t0, t1, t2, t3, t4), (a0, a1, a2, a3, a4),
                (o0, o1, o2, o3, o4), v0s)):
            v128 = (v0 // 128) * 128
            vp = _vp(v0)
            nfull = v128 // CW
            tail = v128 - nfull * CW

            def do_copy(src, dst_off, width, out=out):
                pltpu.sync_copy(src, buf.at[:, pl.ds(0, width)])
                for c in range(ED):
                    pltpu.sync_copy(buf.at[c, pl.ds(0, width)],
                                    out.at[pl.ds(c * _vp(v0) + dst_off,
                                                 width)])

            def do_chunk(m, width, tab=tab):
                do_copy(tab.at[:, pl.ds(m * CW, width)], m * CW, width)

            if nfull >= NW:
                for g in range((nfull + NW - 1) // NW):
                    m = jnp.minimum(wid + NW * g, nfull - 1)
                    do_chunk(m, CW)
            else:
                @pl.when(wid < nfull)
                def _():
                    do_chunk(wid, CW)

            @pl.when(wid == (t % NW))
            def _(ta=ta, v128=v128, tail=tail):
                if tail:
                    do_chunk(nfull, tail)
                do_copy(ta.at[:, :], v128, 128)

    return k(*tts, *tails)


def _gather5(uid2, tid2, tca2, sid2, sca2, tabs):
    mesh = plsc.VectorSubcoreMesh(core_axis_name="c", subcore_axis_name="s")
    out_t = [jax.ShapeDtypeStruct((ED, B), jnp.float32)] * 5
    scratch = ([pltpu.VMEM((BPW,), jnp.int32) for _ in range(5)]
               + [pltpu.VMEM((ED, BPW), jnp.float32) for _ in range(5)]
               + [pltpu.SemaphoreType.DMA])

    @functools.partial(pl.kernel, out_type=out_t, mesh=mesh,
                       scratch_types=scratch,
                       compiler_params=pltpu.CompilerParams(
                           use_tc_tiling_on_sc=False))
    def k(uid_h, tid_h, tca_h, sid_h, sca_h,
          t0, t1, t2, t3, t4,
          o0, o1, o2, o3, o4,
          i0, i1, i2, i3, i4, r0, r1, r2, r3, r4, sem):
        wid = lax.axis_index("s") * NC + lax.axis_index("c")
        idx_hs = (uid_h, tid_h, tca_h, sid_h, sca_h)
        idx_vs = (i0, i1, i2, i3, i4)
        row_vs = (r0, r1, r2, r3, r4)
        tab_hs = (t0, t1, t2, t3, t4)
        outs = (o0, o1, o2, o3, o4)
        for t in range(5):
            pltpu.sync_copy(idx_hs[t].at[pl.ds(wid * BPW, BPW)], idx_vs[t])
        cps = []
        for t in range(5):
            v0 = tab_hs[t].shape[0] // ED
            for c in range(ED):
                cps.append(pltpu.async_copy(
                    tab_hs[t].at[pl.ds(c * v0, v0)].at[idx_vs[t]],
                    row_vs[t].at[c], sem))
        for cp in cps:
            cp.wait()
        for t in range(5):
            pltpu.sync_copy(row_vs[t], outs[t].at[:, pl.ds(wid * BPW, BPW)])

    return k(uid2, tid2, tca2, sid2, sca2, *tabs)


def _mlp_body(eu, ti, tc, si, sc,
              su, wsm, wsh, tu, wtm, wth, hm, hh,
              ws1, h1, wt1, ws2, h2, wt2, ws3, h3, wt3,
              sw, sb, tw, tb, rs, rt):
    d = lambda w, x: lax.dot_general(w[...], x, (((1,), (0,)), ((), ())),
                                     preferred_element_type=jnp.float32)
    eu_, ti_, tc_, si_, sc_ = eu[...], ti[...], tc[...], si[...], sc[...]
    a_s = d(su, eu_) + d(wsm, si_) + d(wsh, sc_) + d(hm, ti_) + d(hh, tc_)
    a_t = d(tu, eu_) + d(wtm, ti_) + d(wth, tc_) + d(hm, si_) + d(hh, sc_)
    xs = jnp.maximum(a_s, 0.0)
    xt = jnp.maximum(a_t, 0.0)
    for (w, h, wt) in ((ws1, h1, wt1), (ws2, h2, wt2), (ws3, h3, wt3)):
        ns = jnp.maximum(d(w, xs) + d(h, xt), 0.0)
        nt = jnp.maximum(d(wt, xt) + d(h, xs), 0.0)
        xs, xt = ns, nt
    ls = d(sw, xs) + sb[...]
    lt = d(tw, xt) + tb[...]
    rs[...] = 1.0 / (1.0 + jnp.exp(-ls))
    rt[...] = 1.0 / (1.0 + jnp.exp(-lt))


def _mlp(eu, ti, tc, si, sc, mats, sw, sb, tw, tb, interpret=False):
    BB = 2048
    grid = (B // BB,)
    dspec = pl.BlockSpec((ED, BB), lambda i: (0, i))
    wspec = lambda a: pl.BlockSpec(a.shape, lambda i: (0, 0))
    in_specs = ([dspec] * 5 + [wspec(m) for m in mats]
                + [wspec(sw), wspec(sb), wspec(tw), wspec(tb)])
    out_specs = [pl.BlockSpec((1, BB), lambda i: (0, i))] * 2
    out_shape = [jax.ShapeDtypeStruct((1, B), jnp.float32)] * 2
    return pl.pallas_call(
        _mlp_body, grid=grid, in_specs=in_specs, out_specs=out_specs,
        out_shape=out_shape, interpret=interpret,
    )(eu, ti, tc, si, sc, *mats, sw, sb, tw, tb)


def kernel(userid, t_can_id, t_can_cate, s_can_id, s_can_cate,
           user_emb, t_itemid_emb, t_itemcate_emb, s_itemid_emb, s_itemcate_emb,
           ws0, h0, wt0, ws1, h1, wt1, ws2, h2, wt2, ws3, h3, wt3,
           s_pred_w, s_pred_b, t_pred_w, t_pred_b):
    tts = [t.T for t in (user_emb, t_itemid_emb, t_itemcate_emb,
                         s_itemid_emb, s_itemcate_emb)]
    tails = [jnp.pad(tt[:, (tt.shape[1] - 1) // 128 * 128:],
                     ((0, 0), (0, 128 - (tt.shape[1]
                                         - (tt.shape[1] - 1) // 128 * 128))))
             for tt in tts]
    tabs = _detile(tts, tails)
    eu, ti, tc, si, sc = _gather5(userid, t_can_id, t_can_cate,
                                  s_can_id, s_can_cate, tabs)
    mats = (ws0[:, :ED] + h0[:, :ED],
            ws0[:, ED:2 * ED], ws0[:, 2 * ED:],
            wt0[:, :ED] + h0[:, :ED],
            wt0[:, ED:2 * ED], wt0[:, 2 * ED:],
            h0[:, ED:2 * ED], h0[:, 2 * ED:],
            ws1, h1, wt1, ws2, h2, wt2, ws3, h3, wt3)
    rs, rt = _mlp(eu, ti, tc, si, sc, mats,
                  s_pred_w, s_pred_b.reshape(1, 1),
                  t_pred_w, t_pred_b.reshape(1, 1))
    return rs.reshape(B), rt.reshape(B)


# speedup vs baseline: 1.0830x; 1.0830x over previous
import functools

import jax
import jax.numpy as jnp
from jax import lax
from jax.experimental import pallas as pl
from jax.experimental.pallas import tpu as pltpu
from jax.experimental.pallas import tpu_sc as plsc

B = 16384
ED = 10
NC, NS = 2, 16
NW = NC * NS
BPW = B // NW
CHUNK = 128
NCH = BPW // CHUNK


CW = 8192


def _vp(v0):
    return -(-v0 // 128) * 128


def _detile_tc(tt):
    v0 = tt.shape[1] - 1
    vp = _vp(v0)
    blk = 32768
    grid = (pl.cdiv(tt.shape[1], blk),)
    out_specs = [pl.BlockSpec((blk,), lambda i: (i,))] * ED
    out_shape = [jax.ShapeDtypeStruct((vp,), jnp.float32)] * ED

    def body(tab, *outs):
        for c in range(ED):
            outs[c][...] = tab[c, :]

    return pl.pallas_call(
        body, grid=grid,
        in_specs=[pl.BlockSpec((ED, blk), lambda i: (0, i))],
        out_specs=out_specs, out_shape=out_shape,
    )(tt)


def _detile(tts, tails):
    mesh = plsc.VectorSubcoreMesh(core_axis_name="c", subcore_axis_name="s")
    v0s = [t.shape[1] - 1 for t in tts]
    out_t = [jax.ShapeDtypeStruct((ED * _vp(v0),), jnp.float32) for v0 in v0s]

    @functools.partial(pl.kernel, out_type=out_t, mesh=mesh,
                       scratch_types=[pltpu.VMEM((ED, CW), jnp.float32)],
                       compiler_params=pltpu.CompilerParams(
                           use_tc_tiling_on_sc=True))
    def k(t1, t2, t3, t4, a1, a2, a3, a4,
          o1, o2, o3, o4, buf):
        wid = lax.axis_index("s") * NC + lax.axis_index("c")
        for t, (tab, ta, out, v0) in enumerate(zip(
                (t1, t2, t3, t4), (a1, a2, a3, a4),
                (o1, o2, o3, o4), v0s)):
            v128 = (v0 // 128) * 128
            vp = _vp(v0)
            nfull = v128 // CW
            tail = v128 - nfull * CW

            def do_copy(src, dst_off, width, out=out):
                pltpu.sync_copy(src, buf.at[:, pl.ds(0, width)])
                for c in range(ED):
                    pltpu.sync_copy(buf.at[c, pl.ds(0, width)],
                                    out.at[pl.ds(c * _vp(v0) + dst_off,
                                                 width)])

            def do_chunk(m, width, tab=tab):
                do_copy(tab.at[:, pl.ds(m * CW, width)], m * CW, width)

            if nfull >= NW:
                for g in range((nfull + NW - 1) // NW):
                    m = jnp.minimum(wid + NW * g, nfull - 1)
                    do_chunk(m, CW)
            else:
                @pl.when(wid < nfull)
                def _():
                    do_chunk(wid, CW)

            @pl.when(wid == (t % NW))
            def _(ta=ta, v128=v128, tail=tail):
                if tail:
                    do_chunk(nfull, tail)
                do_copy(ta.at[:, :], v128, 128)

    return k(*tts, *tails)


def _gather5(uid2, tid2, tca2, sid2, sca2, tabs):
    mesh = plsc.VectorSubcoreMesh(core_axis_name="c", subcore_axis_name="s")
    out_t = [jax.ShapeDtypeStruct((ED, B), jnp.float32)] * 5
    scratch = ([pltpu.VMEM((BPW,), jnp.int32) for _ in range(5)]
               + [pltpu.VMEM((ED, BPW), jnp.float32) for _ in range(5)]
               + [pltpu.SemaphoreType.DMA])

    @functools.partial(pl.kernel, out_type=out_t, mesh=mesh,
                       scratch_types=scratch,
                       compiler_params=pltpu.CompilerParams(
                           use_tc_tiling_on_sc=False))
    def k(uid_h, tid_h, tca_h, sid_h, sca_h,
          u0, u1, u2, u3, u4, u5, u6, u7, u8, u9,
          t1, t2, t3, t4,
          o0, o1, o2, o3, o4,
          i0, i1, i2, i3, i4, r0, r1, r2, r3, r4, sem):
        wid = lax.axis_index("s") * NC + lax.axis_index("c")
        idx_hs = (uid_h, tid_h, tca_h, sid_h, sca_h)
        idx_vs = (i0, i1, i2, i3, i4)
        row_vs = (r0, r1, r2, r3, r4)
        t0cs = (u0, u1, u2, u3, u4, u5, u6, u7, u8, u9)
        tab_hs = (t1, t2, t3, t4)
        outs = (o0, o1, o2, o3, o4)
        for t in range(5):
            pltpu.sync_copy(idx_hs[t].at[pl.ds(wid * BPW, BPW)], idx_vs[t])
        cps = []
        for c in range(ED):
            cps.append(pltpu.async_copy(
                t0cs[c].at[idx_vs[0]], row_vs[0].at[c], sem))
        for t in range(1, 5):
            v0 = tab_hs[t - 1].shape[0] // ED
            for c in range(ED):
                cps.append(pltpu.async_copy(
                    tab_hs[t - 1].at[pl.ds(c * v0, v0)].at[idx_vs[t]],
                    row_vs[t].at[c], sem))
        for cp in cps:
            cp.wait()
        for t in range(5):
            pltpu.sync_copy(row_vs[t], outs[t].at[:, pl.ds(wid * BPW, BPW)])

    return k(uid2, tid2, tca2, sid2, sca2, *tabs)


def _mlp_body(eu, ti, tc, si, sc,
              su, wsm, wsh, tu, wtm, wth, hm, hh,
              ws1, h1, wt1, ws2, h2, wt2, ws3, h3, wt3,
              sw, sb, tw, tb, rs, rt):
    d = lambda w, x: lax.dot_general(w[...], x, (((1,), (0,)), ((), ())),
                                     preferred_element_type=jnp.float32)
    eu_, ti_, tc_, si_, sc_ = eu[...], ti[...], tc[...], si[...], sc[...]
    a_s = d(su, eu_) + d(wsm, si_) + d(wsh, sc_) + d(hm, ti_) + d(hh, tc_)
    a_t = d(tu, eu_) + d(wtm, ti_) + d(wth, tc_) + d(hm, si_) + d(hh, sc_)
    xs = jnp.maximum(a_s, 0.0)
    xt = jnp.maximum(a_t, 0.0)
    for (w, h, wt) in ((ws1, h1, wt1), (ws2, h2, wt2), (ws3, h3, wt3)):
        ns = jnp.maximum(d(w, xs) + d(h, xt), 0.0)
        nt = jnp.maximum(d(wt, xt) + d(h, xs), 0.0)
        xs, xt = ns, nt
    ls = d(sw, xs) + sb[...]
    lt = d(tw, xt) + tb[...]
    rs[...] = 1.0 / (1.0 + jnp.exp(-ls))
    rt[...] = 1.0 / (1.0 + jnp.exp(-lt))


def _mlp(eu, ti, tc, si, sc, mats, sw, sb, tw, tb, interpret=False):
    BB = 2048
    grid = (B // BB,)
    dspec = pl.BlockSpec((ED, BB), lambda i: (0, i))
    wspec = lambda a: pl.BlockSpec(a.shape, lambda i: (0, 0))
    in_specs = ([dspec] * 5 + [wspec(m) for m in mats]
                + [wspec(sw), wspec(sb), wspec(tw), wspec(tb)])
    out_specs = [pl.BlockSpec((1, BB), lambda i: (0, i))] * 2
    out_shape = [jax.ShapeDtypeStruct((1, B), jnp.float32)] * 2
    return pl.pallas_call(
        _mlp_body, grid=grid, in_specs=in_specs, out_specs=out_specs,
        out_shape=out_shape, interpret=interpret,
    )(eu, ti, tc, si, sc, *mats, sw, sb, tw, tb)


def kernel(userid, t_can_id, t_can_cate, s_can_id, s_can_cate,
           user_emb, t_itemid_emb, t_itemcate_emb, s_itemid_emb, s_itemcate_emb,
           ws0, h0, wt0, ws1, h1, wt1, ws2, h2, wt2, ws3, h3, wt3,
           s_pred_w, s_pred_b, t_pred_w, t_pred_b):
    tts = [t.T for t in (user_emb, t_itemid_emb, t_itemcate_emb,
                         s_itemid_emb, s_itemcate_emb)]
    tails = [jnp.pad(tt[:, (tt.shape[1] - 1) // 128 * 128:],
                     ((0, 0), (0, 128 - (tt.shape[1]
                                         - (tt.shape[1] - 1) // 128 * 128))))
             for tt in tts[1:]]
    tabs = list(_detile_tc(tts[0])) + list(_detile(tts[1:], tails))
    eu, ti, tc, si, sc = _gather5(userid, t_can_id, t_can_cate,
                                  s_can_id, s_can_cate, tabs)
    mats = (ws0[:, :ED] + h0[:, :ED],
            ws0[:, ED:2 * ED], ws0[:, 2 * ED:],
            wt0[:, :ED] + h0[:, :ED],
            wt0[:, ED:2 * ED], wt0[:, 2 * ED:],
            h0[:, ED:2 * ED], h0[:, 2 * ED:],
            ws1, h1, wt1, ws2, h2, wt2, ws3, h3, wt3)
    rs, rt = _mlp(eu, ti, tc, si, sc, mats,
                  s_pred_w, s_pred_b.reshape(1, 1),
                  t_pred_w, t_pred_b.reshape(1, 1))
    return rs.reshape(B), rt.reshape(B)


# speedup vs baseline: 1.1069x; 1.0221x over previous
import functools

import jax
import jax.numpy as jnp
from jax import lax
from jax.experimental import pallas as pl
from jax.experimental.pallas import tpu as pltpu
from jax.experimental.pallas import tpu_sc as plsc

B = 16384
ED = 10
NC, NS = 2, 16
NW = NC * NS
BPW = B // NW
CHUNK = 128
NCH = BPW // CHUNK


CW = 8192


def _vp(v0):
    return -(-v0 // 128) * 128


def _detile_tc(tt):
    v0 = tt.shape[1] - 1
    vp = _vp(v0)
    blk = 32768
    grid = (pl.cdiv(tt.shape[1], blk),)
    out_specs = [pl.BlockSpec((blk,), lambda i: (i,))] * ED
    out_shape = [jax.ShapeDtypeStruct((vp,), jnp.float32)] * ED

    def body(tab, *outs):
        for c in range(ED):
            outs[c][...] = tab[c, :]

    return pl.pallas_call(
        body, grid=grid,
        in_specs=[pl.BlockSpec((ED, blk), lambda i: (0, i))],
        out_specs=out_specs, out_shape=out_shape,
    )(tt)


def _detile(tts, tails):
    mesh = plsc.VectorSubcoreMesh(core_axis_name="c", subcore_axis_name="s")
    v0s = [t.shape[1] - 1 for t in tts]
    out_t = [jax.ShapeDtypeStruct((ED * _vp(v0),), jnp.float32) for v0 in v0s]

    @functools.partial(pl.kernel, out_type=out_t, mesh=mesh,
                       scratch_types=[pltpu.VMEM((ED, CW), jnp.float32),
                                      pltpu.SemaphoreType.DMA],
                       compiler_params=pltpu.CompilerParams(
                           use_tc_tiling_on_sc=True))
    def k(t1, t2, t3, t4, a1, a2, a3, a4,
          o1, o2, o3, o4, buf, wsem):
        wid = lax.axis_index("s") * NC + lax.axis_index("c")
        for t, (tab, ta, out, v0) in enumerate(zip(
                (t1, t2, t3, t4), (a1, a2, a3, a4),
                (o1, o2, o3, o4), v0s)):
            v128 = (v0 // 128) * 128
            vp = _vp(v0)
            nfull = v128 // CW
            tail = v128 - nfull * CW

            def do_copy(src, dst_off, width, out=out):
                pltpu.sync_copy(src, buf.at[:, pl.ds(0, width)])
                ws = [pltpu.async_copy(
                    buf.at[c, pl.ds(0, width)],
                    out.at[pl.ds(c * _vp(v0) + dst_off, width)], wsem)
                    for c in range(ED)]
                for w in ws:
                    w.wait()

            def do_chunk(m, width, tab=tab):
                do_copy(tab.at[:, pl.ds(m * CW, width)], m * CW, width)

            if nfull >= NW:
                for g in range((nfull + NW - 1) // NW):
                    m = jnp.minimum(wid + NW * g, nfull - 1)
                    do_chunk(m, CW)
            else:
                @pl.when(wid < nfull)
                def _():
                    do_chunk(wid, CW)

            @pl.when(wid == (t % NW))
            def _(ta=ta, v128=v128, tail=tail):
                if tail:
                    do_chunk(nfull, tail)
                do_copy(ta.at[:, :], v128, 128)

    return k(*tts, *tails)


def _gather5(uid2, tid2, tca2, sid2, sca2, tabs):
    mesh = plsc.VectorSubcoreMesh(core_axis_name="c", subcore_axis_name="s")
    out_t = [jax.ShapeDtypeStruct((ED, B), jnp.float32)] * 5
    scratch = ([pltpu.VMEM((BPW,), jnp.int32) for _ in range(5)]
               + [pltpu.VMEM((ED, BPW), jnp.float32) for _ in range(5)]
               + [pltpu.SemaphoreType.DMA])

    @functools.partial(pl.kernel, out_type=out_t, mesh=mesh,
                       scratch_types=scratch,
                       compiler_params=pltpu.CompilerParams(
                           use_tc_tiling_on_sc=False))
    def k(uid_h, tid_h, tca_h, sid_h, sca_h,
          u0, u1, u2, u3, u4, u5, u6, u7, u8, u9,
          t1, t2, t3, t4,
          o0, o1, o2, o3, o4,
          i0, i1, i2, i3, i4, r0, r1, r2, r3, r4, sem):
        wid = lax.axis_index("s") * NC + lax.axis_index("c")
        idx_hs = (uid_h, tid_h, tca_h, sid_h, sca_h)
        idx_vs = (i0, i1, i2, i3, i4)
        row_vs = (r0, r1, r2, r3, r4)
        t0cs = (u0, u1, u2, u3, u4, u5, u6, u7, u8, u9)
        tab_hs = (t1, t2, t3, t4)
        outs = (o0, o1, o2, o3, o4)
        for t in range(5):
            pltpu.sync_copy(idx_hs[t].at[pl.ds(wid * BPW, BPW)], idx_vs[t])
        cps = []
        for c in range(ED):
            cps.append(pltpu.async_copy(
                t0cs[c].at[idx_vs[0]], row_vs[0].at[c], sem))
        for t in range(1, 5):
            v0 = tab_hs[t - 1].shape[0] // ED
            for c in range(ED):
                cps.append(pltpu.async_copy(
                    tab_hs[t - 1].at[pl.ds(c * v0, v0)].at[idx_vs[t]],
                    row_vs[t].at[c], sem))
        for cp in cps:
            cp.wait()
        for t in range(5):
            pltpu.sync_copy(row_vs[t], outs[t].at[:, pl.ds(wid * BPW, BPW)])

    return k(uid2, tid2, tca2, sid2, sca2, *tabs)


def _mlp_body(eu, ti, tc, si, sc,
              su, wsm, wsh, tu, wtm, wth, hm, hh,
              ws1, h1, wt1, ws2, h2, wt2, ws3, h3, wt3,
              sw, sb, tw, tb, rs, rt):
    d = lambda w, x: lax.dot_general(w[...], x, (((1,), (0,)), ((), ())),
                                     preferred_element_type=jnp.float32)
    eu_, ti_, tc_, si_, sc_ = eu[...], ti[...], tc[...], si[...], sc[...]
    a_s = d(su, eu_) + d(wsm, si_) + d(wsh, sc_) + d(hm, ti_) + d(hh, tc_)
    a_t = d(tu, eu_) + d(wtm, ti_) + d(wth, tc_) + d(hm, si_) + d(hh, sc_)
    xs = jnp.maximum(a_s, 0.0)
    xt = jnp.maximum(a_t, 0.0)
    for (w, h, wt) in ((ws1, h1, wt1), (ws2, h2, wt2), (ws3, h3, wt3)):
        ns = jnp.maximum(d(w, xs) + d(h, xt), 0.0)
        nt = jnp.maximum(d(wt, xt) + d(h, xs), 0.0)
        xs, xt = ns, nt
    ls = d(sw, xs) + sb[...]
    lt = d(tw, xt) + tb[...]
    rs[...] = 1.0 / (1.0 + jnp.exp(-ls))
    rt[...] = 1.0 / (1.0 + jnp.exp(-lt))


def _mlp(eu, ti, tc, si, sc, mats, sw, sb, tw, tb, interpret=False):
    BB = 2048
    grid = (B // BB,)
    dspec = pl.BlockSpec((ED, BB), lambda i: (0, i))
    wspec = lambda a: pl.BlockSpec(a.shape, lambda i: (0, 0))
    in_specs = ([dspec] * 5 + [wspec(m) for m in mats]
                + [wspec(sw), wspec(sb), wspec(tw), wspec(tb)])
    out_specs = [pl.BlockSpec((1, BB), lambda i: (0, i))] * 2
    out_shape = [jax.ShapeDtypeStruct((1, B), jnp.float32)] * 2
    return pl.pallas_call(
        _mlp_body, grid=grid, in_specs=in_specs, out_specs=out_specs,
        out_shape=out_shape, interpret=interpret,
    )(eu, ti, tc, si, sc, *mats, sw, sb, tw, tb)


def kernel(userid, t_can_id, t_can_cate, s_can_id, s_can_cate,
           user_emb, t_itemid_emb, t_itemcate_emb, s_itemid_emb, s_itemcate_emb,
           ws0, h0, wt0, ws1, h1, wt1, ws2, h2, wt2, ws3, h3, wt3,
           s_pred_w, s_pred_b, t_pred_w, t_pred_b):
    tts = [t.T for t in (user_emb, t_itemid_emb, t_itemcate_emb,
                         s_itemid_emb, s_itemcate_emb)]
    tails = [jnp.pad(tt[:, (tt.shape[1] - 1) // 128 * 128:],
                     ((0, 0), (0, 128 - (tt.shape[1]
                                         - (tt.shape[1] - 1) // 128 * 128))))
             for tt in tts[1:]]
    tabs = list(_detile_tc(tts[0])) + list(_detile(tts[1:], tails))
    eu, ti, tc, si, sc = _gather5(userid, t_can_id, t_can_cate,
                                  s_can_id, s_can_cate, tabs)
    mats = (ws0[:, :ED] + h0[:, :ED],
            ws0[:, ED:2 * ED], ws0[:, 2 * ED:],
            wt0[:, :ED] + h0[:, :ED],
            wt0[:, ED:2 * ED], wt0[:, 2 * ED:],
            h0[:, ED:2 * ED], h0[:, 2 * ED:],
            ws1, h1, wt1, ws2, h2, wt2, ws3, h3, wt3)
    rs, rt = _mlp(eu, ti, tc, si, sc, mats,
                  s_pred_w, s_pred_b.reshape(1, 1),
                  t_pred_w, t_pred_b.reshape(1, 1))
    return rs.reshape(B), rt.reshape(B)


# speedup vs baseline: 1.1784x; 1.0646x over previous
import functools

import jax
import jax.numpy as jnp
from jax import lax
from jax.experimental import pallas as pl
from jax.experimental.pallas import tpu as pltpu
from jax.experimental.pallas import tpu_sc as plsc

B = 16384
ED = 10
NC, NS = 2, 16
NW = NC * NS
BPW = B // NW
CHUNK = 128
NCH = BPW // CHUNK


CW = 8192


def _vp(v0):
    return -(-v0 // 128) * 128


def _detile_tc(tt):
    v0 = tt.shape[1] - 1
    vp = _vp(v0)
    blk = 32768
    grid = (pl.cdiv(tt.shape[1], blk),)
    out_specs = [pl.BlockSpec((blk,), lambda i: (i,))] * ED
    out_shape = [jax.ShapeDtypeStruct((vp,), jnp.float32)] * ED

    def body(tab, *outs):
        for c in range(ED):
            outs[c][...] = tab[c, :]

    return pl.pallas_call(
        body, grid=grid,
        in_specs=[pl.BlockSpec((ED, blk), lambda i: (0, i))],
        out_specs=out_specs, out_shape=out_shape,
    )(tt)


def _detile(tts, tails):
    mesh = plsc.VectorSubcoreMesh(core_axis_name="c", subcore_axis_name="s")
    v0s = [t.shape[1] - 1 for t in tts]
    out_t = [jax.ShapeDtypeStruct((_vp(v0),), jnp.float32)
             for v0 in v0s for _ in range(ED)]

    @functools.partial(pl.kernel, out_type=out_t, mesh=mesh,
                       scratch_types=[pltpu.VMEM((ED, CW), jnp.float32),
                                      pltpu.SemaphoreType.DMA],
                       compiler_params=pltpu.CompilerParams(
                           use_tc_tiling_on_sc=True))
    def k(*refs):
        nt = len(tts)
        tabs, tails_r = refs[:nt], refs[nt:2 * nt]
        out_r, (buf, wsem) = refs[2 * nt:-2], refs[-2:]
        wid = lax.axis_index("s") * NC + lax.axis_index("c")
        for t, (tab, ta, v0) in enumerate(zip(tabs, tails_r, v0s)):
            outs = out_r[t * ED:(t + 1) * ED]
            v128 = (v0 // 128) * 128
            nfull = v128 // CW
            tail = v128 - nfull * CW

            def do_copy(src, dst_off, width, outs=outs):
                pltpu.sync_copy(src, buf.at[:, pl.ds(0, width)])
                ws = [pltpu.async_copy(
                    buf.at[c, pl.ds(0, width)],
                    outs[c].at[pl.ds(dst_off, width)], wsem)
                    for c in range(ED)]
                for w in ws:
                    w.wait()

            def do_chunk(m, width, tab=tab):
                do_copy(tab.at[:, pl.ds(m * CW, width)], m * CW, width)

            if nfull >= NW:
                for g in range((nfull + NW - 1) // NW):
                    m = jnp.minimum(wid + NW * g, nfull - 1)
                    do_chunk(m, CW)
            else:
                @pl.when(wid < nfull)
                def _():
                    do_chunk(wid, CW)

            @pl.when(wid == (t % NW))
            def _(ta=ta, v128=v128, tail=tail):
                if tail:
                    do_chunk(nfull, tail)
                do_copy(ta.at[:, :], v128, 128)

    return k(*tts, *tails)


def _gather5(uid2, tid2, tca2, sid2, sca2, tabs):
    mesh = plsc.VectorSubcoreMesh(core_axis_name="c", subcore_axis_name="s")
    out_t = [jax.ShapeDtypeStruct((ED, B), jnp.float32)] * 5
    scratch = ([pltpu.VMEM((BPW,), jnp.int32) for _ in range(5)]
               + [pltpu.VMEM((ED, BPW), jnp.float32) for _ in range(5)]
               + [pltpu.SemaphoreType.DMA])

    @functools.partial(pl.kernel, out_type=out_t, mesh=mesh,
                       scratch_types=scratch,
                       compiler_params=pltpu.CompilerParams(
                           use_tc_tiling_on_sc=False))
    def k(*refs):
        idx_hs = refs[:5]
        tab_cs = refs[5:55]
        outs = refs[55:60]
        idx_vs = refs[60:65]
        row_vs = refs[65:70]
        sem = refs[70]
        wid = lax.axis_index("s") * NC + lax.axis_index("c")
        for t in range(5):
            pltpu.sync_copy(idx_hs[t].at[pl.ds(wid * BPW, BPW)], idx_vs[t])
        cps = []
        for t in range(5):
            for c in range(ED):
                cps.append(pltpu.async_copy(
                    tab_cs[t * ED + c].at[idx_vs[t]], row_vs[t].at[c], sem))
        for cp in cps:
            cp.wait()
        for t in range(5):
            pltpu.sync_copy(row_vs[t], outs[t].at[:, pl.ds(wid * BPW, BPW)])

    return k(uid2, tid2, tca2, sid2, sca2, *tabs)


def _mlp_body(eu, ti, tc, si, sc,
              su, wsm, wsh, tu, wtm, wth, hm, hh,
              ws1, h1, wt1, ws2, h2, wt2, ws3, h3, wt3,
              sw, sb, tw, tb, rs, rt):
    d = lambda w, x: lax.dot_general(w[...], x, (((1,), (0,)), ((), ())),
                                     preferred_element_type=jnp.float32)
    eu_, ti_, tc_, si_, sc_ = eu[...], ti[...], tc[...], si[...], sc[...]
    a_s = d(su, eu_) + d(wsm, si_) + d(wsh, sc_) + d(hm, ti_) + d(hh, tc_)
    a_t = d(tu, eu_) + d(wtm, ti_) + d(wth, tc_) + d(hm, si_) + d(hh, sc_)
    xs = jnp.maximum(a_s, 0.0)
    xt = jnp.maximum(a_t, 0.0)
    for (w, h, wt) in ((ws1, h1, wt1), (ws2, h2, wt2), (ws3, h3, wt3)):
        ns = jnp.maximum(d(w, xs) + d(h, xt), 0.0)
        nt = jnp.maximum(d(wt, xt) + d(h, xs), 0.0)
        xs, xt = ns, nt
    ls = d(sw, xs) + sb[...]
    lt = d(tw, xt) + tb[...]
    rs[...] = 1.0 / (1.0 + jnp.exp(-ls))
    rt[...] = 1.0 / (1.0 + jnp.exp(-lt))


def _mlp(eu, ti, tc, si, sc, mats, sw, sb, tw, tb, interpret=False):
    BB = 2048
    grid = (B // BB,)
    dspec = pl.BlockSpec((ED, BB), lambda i: (0, i))
    wspec = lambda a: pl.BlockSpec(a.shape, lambda i: (0, 0))
    in_specs = ([dspec] * 5 + [wspec(m) for m in mats]
                + [wspec(sw), wspec(sb), wspec(tw), wspec(tb)])
    out_specs = [pl.BlockSpec((1, BB), lambda i: (0, i))] * 2
    out_shape = [jax.ShapeDtypeStruct((1, B), jnp.float32)] * 2
    return pl.pallas_call(
        _mlp_body, grid=grid, in_specs=in_specs, out_specs=out_specs,
        out_shape=out_shape, interpret=interpret,
    )(eu, ti, tc, si, sc, *mats, sw, sb, tw, tb)


def kernel(userid, t_can_id, t_can_cate, s_can_id, s_can_cate,
           user_emb, t_itemid_emb, t_itemcate_emb, s_itemid_emb, s_itemcate_emb,
           ws0, h0, wt0, ws1, h1, wt1, ws2, h2, wt2, ws3, h3, wt3,
           s_pred_w, s_pred_b, t_pred_w, t_pred_b):
    tts = [t.T for t in (user_emb, t_itemid_emb, t_itemcate_emb,
                         s_itemid_emb, s_itemcate_emb)]
    sc_tts = [tts[1], tts[3]]
    tails = [jnp.pad(tt[:, (tt.shape[1] - 1) // 128 * 128:],
                     ((0, 0), (0, 128 - (tt.shape[1]
                                         - (tt.shape[1] - 1) // 128 * 128))))
             for tt in sc_tts]
    sc_flat = list(_detile(sc_tts, tails))
    tabs = (list(_detile_tc(tts[0])) + sc_flat[:ED]
            + list(_detile_tc(tts[2])) + sc_flat[ED:]
            + list(_detile_tc(tts[4])))
    eu, ti, tc, si, sc = _gather5(userid, t_can_id, t_can_cate,
                                  s_can_id, s_can_cate, tabs)
    mats = (ws0[:, :ED] + h0[:, :ED],
            ws0[:, ED:2 * ED], ws0[:, 2 * ED:],
            wt0[:, :ED] + h0[:, :ED],
            wt0[:, ED:2 * ED], wt0[:, 2 * ED:],
            h0[:, ED:2 * ED], h0[:, 2 * ED:],
            ws1, h1, wt1, ws2, h2, wt2, ws3, h3, wt3)
    rs, rt = _mlp(eu, ti, tc, si, sc, mats,
                  s_pred_w, s_pred_b.reshape(1, 1),
                  t_pred_w, t_pred_b.reshape(1, 1))
    return rs.reshape(B), rt.reshape(B)


# speedup vs baseline: 1.1912x; 1.0108x over previous
import functools

import jax
import jax.numpy as jnp
from jax import lax
from jax.experimental import pallas as pl
from jax.experimental.pallas import tpu as pltpu
from jax.experimental.pallas import tpu_sc as plsc

B = 16384
ED = 10
NC, NS = 2, 16
NW = NC * NS
BPW = B // NW
CHUNK = 128
NCH = BPW // CHUNK


CW = 4096


def _vp(v0):
    return -(-v0 // 128) * 128


def _detile_tc(tt):
    v0 = tt.shape[1] - 1
    vp = _vp(v0)
    blk = 32768
    grid = (pl.cdiv(tt.shape[1], blk),)
    out_specs = [pl.BlockSpec((blk,), lambda i: (i,))] * ED
    out_shape = [jax.ShapeDtypeStruct((vp,), jnp.float32)] * ED

    def body(tab, *outs):
        for c in range(ED):
            outs[c][...] = tab[c, :]

    return pl.pallas_call(
        body, grid=grid,
        in_specs=[pl.BlockSpec((ED, blk), lambda i: (0, i))],
        out_specs=out_specs, out_shape=out_shape,
    )(tt)


def _detile(tts, tails):
    mesh = plsc.VectorSubcoreMesh(core_axis_name="c", subcore_axis_name="s")
    v0s = [t.shape[1] - 1 for t in tts]
    out_t = [jax.ShapeDtypeStruct((_vp(v0),), jnp.float32)
             for v0 in v0s for _ in range(ED)]

    @functools.partial(pl.kernel, out_type=out_t, mesh=mesh,
                       scratch_types=[pltpu.VMEM((ED, CW), jnp.float32),
                                      pltpu.VMEM((ED, CW), jnp.float32),
                                      pltpu.SemaphoreType.DMA,
                                      pltpu.SemaphoreType.DMA,
                                      pltpu.SemaphoreType.DMA],
                       compiler_params=pltpu.CompilerParams(
                           use_tc_tiling_on_sc=True))
    def k(*refs):
        nt = len(tts)
        tabs, tails_r = refs[:nt], refs[nt:2 * nt]
        out_r = refs[2 * nt:-5]
        buf0, buf1, rsem0, rsem1, wsem = refs[-5:]
        bufs, rsems = (buf0, buf1), (rsem0, rsem1)
        wid = lax.axis_index("s") * NC + lax.axis_index("c")

        jobs = []
        for t, (tab, v0) in enumerate(zip(tabs, v0s)):
            nfull = ((v0 // 128) * 128) // CW
            for g in range((nfull + NW - 1) // NW):
                m = jnp.minimum(wid + NW * g, nfull - 1)
                jobs.append((tab, m, out_r[t * ED:(t + 1) * ED]))

        def fire_read(j):
            tab, m, _ = jobs[j]
            return pltpu.async_copy(tab.at[:, pl.ds(m * CW, CW)],
                                    bufs[j % 2], rsems[j % 2])

        rd = {0: fire_read(0), 1: fire_read(1)}
        for j, (tab, m, outs) in enumerate(jobs):
            rd[j].wait()
            buf = bufs[j % 2]
            ws = [pltpu.async_copy(buf.at[c], outs[c].at[pl.ds(m * CW, CW)],
                                   wsem) for c in range(ED)]
            for w in ws:
                w.wait()
            if j + 2 < len(jobs):
                rd[j + 2] = fire_read(j + 2)

        for t, (tab, ta, v0) in enumerate(zip(tabs, tails_r, v0s)):
            outs = out_r[t * ED:(t + 1) * ED]
            v128 = (v0 // 128) * 128
            nfull = v128 // CW
            tail = v128 - nfull * CW

            @pl.when(wid == (t % NW))
            def _(tab=tab, ta=ta, outs=outs, v128=v128, nfull=nfull,
                  tail=tail):
                def do_copy(src, dst_off, width):
                    pltpu.sync_copy(src, buf0.at[:, pl.ds(0, width)])
                    ws = [pltpu.async_copy(
                        buf0.at[c, pl.ds(0, width)],
                        outs[c].at[pl.ds(dst_off, width)], wsem)
                        for c in range(ED)]
                    for w in ws:
                        w.wait()

                if tail:
                    do_copy(tab.at[:, pl.ds(nfull * CW, tail)],
                            nfull * CW, tail)
                do_copy(ta.at[:, :], v128, 128)

    return k(*tts, *tails)


def _gather5(uid2, tid2, tca2, sid2, sca2, tabs):
    mesh = plsc.VectorSubcoreMesh(core_axis_name="c", subcore_axis_name="s")
    out_t = [jax.ShapeDtypeStruct((ED, B), jnp.float32)] * 5
    scratch = ([pltpu.VMEM((BPW,), jnp.int32) for _ in range(5)]
               + [pltpu.VMEM((ED, BPW), jnp.float32) for _ in range(5)]
               + [pltpu.SemaphoreType.DMA])

    @functools.partial(pl.kernel, out_type=out_t, mesh=mesh,
                       scratch_types=scratch,
                       compiler_params=pltpu.CompilerParams(
                           use_tc_tiling_on_sc=False))
    def k(*refs):
        idx_hs = refs[:5]
        tab_cs = refs[5:55]
        outs = refs[55:60]
        idx_vs = refs[60:65]
        row_vs = refs[65:70]
        sem = refs[70]
        wid = lax.axis_index("s") * NC + lax.axis_index("c")
        for t in range(5):
            pltpu.sync_copy(idx_hs[t].at[pl.ds(wid * BPW, BPW)], idx_vs[t])
        cps = []
        for t in range(5):
            for c in range(ED):
                cps.append(pltpu.async_copy(
                    tab_cs[t * ED + c].at[idx_vs[t]], row_vs[t].at[c], sem))
        for cp in cps:
            cp.wait()
        for t in range(5):
            pltpu.sync_copy(row_vs[t], outs[t].at[:, pl.ds(wid * BPW, BPW)])

    return k(uid2, tid2, tca2, sid2, sca2, *tabs)


def _mlp_body(eu, ti, tc, si, sc,
              su, wsm, wsh, tu, wtm, wth, hm, hh,
              ws1, h1, wt1, ws2, h2, wt2, ws3, h3, wt3,
              sw, sb, tw, tb, rs, rt):
    d = lambda w, x: lax.dot_general(w[...], x, (((1,), (0,)), ((), ())),
                                     preferred_element_type=jnp.float32)
    eu_, ti_, tc_, si_, sc_ = eu[...], ti[...], tc[...], si[...], sc[...]
    a_s = d(su, eu_) + d(wsm, si_) + d(wsh, sc_) + d(hm, ti_) + d(hh, tc_)
    a_t = d(tu, eu_) + d(wtm, ti_) + d(wth, tc_) + d(hm, si_) + d(hh, sc_)
    xs = jnp.maximum(a_s, 0.0)
    xt = jnp.maximum(a_t, 0.0)
    for (w, h, wt) in ((ws1, h1, wt1), (ws2, h2, wt2), (ws3, h3, wt3)):
        ns = jnp.maximum(d(w, xs) + d(h, xt), 0.0)
        nt = jnp.maximum(d(wt, xt) + d(h, xs), 0.0)
        xs, xt = ns, nt
    ls = d(sw, xs) + sb[...]
    lt = d(tw, xt) + tb[...]
    rs[...] = 1.0 / (1.0 + jnp.exp(-ls))
    rt[...] = 1.0 / (1.0 + jnp.exp(-lt))


def _mlp(eu, ti, tc, si, sc, mats, sw, sb, tw, tb, interpret=False):
    BB = 2048
    grid = (B // BB,)
    dspec = pl.BlockSpec((ED, BB), lambda i: (0, i))
    wspec = lambda a: pl.BlockSpec(a.shape, lambda i: (0, 0))
    in_specs = ([dspec] * 5 + [wspec(m) for m in mats]
                + [wspec(sw), wspec(sb), wspec(tw), wspec(tb)])
    out_specs = [pl.BlockSpec((1, BB), lambda i: (0, i))] * 2
    out_shape = [jax.ShapeDtypeStruct((1, B), jnp.float32)] * 2
    return pl.pallas_call(
        _mlp_body, grid=grid, in_specs=in_specs, out_specs=out_specs,
        out_shape=out_shape, interpret=interpret,
    )(eu, ti, tc, si, sc, *mats, sw, sb, tw, tb)


def kernel(userid, t_can_id, t_can_cate, s_can_id, s_can_cate,
           user_emb, t_itemid_emb, t_itemcate_emb, s_itemid_emb, s_itemcate_emb,
           ws0, h0, wt0, ws1, h1, wt1, ws2, h2, wt2, ws3, h3, wt3,
           s_pred_w, s_pred_b, t_pred_w, t_pred_b):
    tts = [t.T for t in (user_emb, t_itemid_emb, t_itemcate_emb,
                         s_itemid_emb, s_itemcate_emb)]
    sc_tts = [tts[1], tts[3]]
    tails = [jnp.pad(tt[:, (tt.shape[1] - 1) // 128 * 128:],
                     ((0, 0), (0, 128 - (tt.shape[1]
                                         - (tt.shape[1] - 1) // 128 * 128))))
             for tt in sc_tts]
    sc_flat = list(_detile(sc_tts, tails))
    tabs = (list(_detile_tc(tts[0])) + sc_flat[:ED]
            + list(_detile_tc(tts[2])) + sc_flat[ED:]
            + list(_detile_tc(tts[4])))
    eu, ti, tc, si, sc = _gather5(userid, t_can_id, t_can_cate,
                                  s_can_id, s_can_cate, tabs)
    mats = (ws0[:, :ED] + h0[:, :ED],
            ws0[:, ED:2 * ED], ws0[:, 2 * ED:],
            wt0[:, :ED] + h0[:, :ED],
            wt0[:, ED:2 * ED], wt0[:, 2 * ED:],
            h0[:, ED:2 * ED], h0[:, 2 * ED:],
            ws1, h1, wt1, ws2, h2, wt2, ws3, h3, wt3)
    rs, rt = _mlp(eu, ti, tc, si, sc, mats,
                  s_pred_w, s_pred_b.reshape(1, 1),
                  t_pred_w, t_pred_b.reshape(1, 1))
    return rs.reshape(B), rt.reshape(B)
